# baseline (speedup 1.0000x reference)
import functools
import math

import jax
import jax.numpy as jnp
import numpy as np
from jax import lax
from jax.experimental import pallas as pl
from jax.experimental.pallas import tpu as pltpu
from jax.experimental.pallas import tpu_sc as plsc

_EPS = 1e-5
_LANES = 16


_GATHER_DNUMS = lax.GatherDimensionNumbers(
    offset_dims=(), collapsed_slice_dims=(0,), start_index_map=(0,))


def _lanes_sum(x):
    idx = lax.iota(jnp.int32, _LANES)
    for sh in (8, 4, 2, 1):
        perm = idx ^ sh
        x = x + lax.gather(x, perm[:, None], _GATHER_DNUMS, slice_sizes=(1,),
                           mode=lax.GatherScatterMode.PROMISE_IN_BOUNDS)
    return x


def _pe_const(seq_len, dim):
    position = np.arange(seq_len, dtype=np.float32)[:, None]
    div_term = np.exp(
        np.arange(0, dim, 2, dtype=np.float32) * -(math.log(10000.0) / dim))
    pe = np.zeros((seq_len, dim), dtype=np.float32)
    pe[:, 0::2] = np.sin(position * div_term)
    pe[:, 1::2] = np.cos(position * div_term)
    return pe


def _name_branch_sc(idx_grp, table, pe, gamma, beta):
    NW, NCH, T = idx_grp.shape
    V, H = table.shape
    S = pe.shape[0]
    info = plsc.get_sparse_core_info()
    NC = info.num_cores
    per_w = NCH * T
    mesh = plsc.VectorSubcoreMesh(core_axis_name="c", subcore_axis_name="s")

    @functools.partial(
        pl.kernel,
        mesh=mesh,
        out_type=jax.ShapeDtypeStruct((NW * per_w, H), jnp.float32),
        scratch_types=[
            pltpu.VMEM((NCH, T), jnp.int32),
            pltpu.VMEM((T, H), jnp.float32),
            pltpu.VMEM((T, H), jnp.float32),
            pltpu.VMEM((H,), jnp.float32),
            pltpu.VMEM((H,), jnp.float32),
            pltpu.SemaphoreType.DMA,
        ],
    )
    def k(idx_hbm, table_hbm, pe_hbm, gamma_hbm, beta_hbm, out_hbm,
          idx_v, rows_v, pe_v, g_v, b_v, sem):
        wid = lax.axis_index("s") * NC + lax.axis_index("c")
        base = wid * per_w
        s0 = lax.rem(base, S)
        pltpu.sync_copy(idx_hbm.at[wid], idx_v)
        pltpu.sync_copy(gamma_hbm, g_v)
        pltpu.sync_copy(beta_hbm, b_v)

        def chunk(j, carry):
            pltpu.async_copy(table_hbm.at[idx_v.at[j]], rows_v, sem).wait()
            pltpu.sync_copy(pe_hbm.at[pl.ds(s0 + j * T, T)], pe_v)

            def row(i, c):
                sum_v = jnp.zeros((_LANES,), jnp.float32)
                sq_v = jnp.zeros((_LANES,), jnp.float32)
                for kk in range(H // _LANES):
                    sl = pl.ds(kk * _LANES, _LANES)
                    x = rows_v[i, sl] + pe_v[i, sl]
                    rows_v[i, sl] = x
                    sum_v = sum_v + x
                    sq_v = sq_v + x * x
                mu_v = _lanes_sum(sum_v) * (1.0 / H)
                var_v = _lanes_sum(sq_v) * (1.0 / H) - mu_v * mu_v
                v = var_v + _EPS
                yi = (jnp.int32(0x5F3759DF)
                      - (lax.bitcast_convert_type(v, jnp.int32) >> 1))
                y = lax.bitcast_convert_type(yi, jnp.float32)
                for _ in range(3):
                    y = y * (1.5 - 0.5 * v * y * y)
                for kk in range(H // _LANES):
                    sl = pl.ds(kk * _LANES, _LANES)
                    x = (rows_v[i, sl] - mu_v) * y
                    rows_v[i, sl] = x * g_v[sl] + b_v[sl]
                return c

            lax.fori_loop(0, T, row, 0)
            pltpu.sync_copy(rows_v, out_hbm.at[pl.ds(base + j * T, T)])
            return carry

        lax.fori_loop(0, NCH, chunk, 0)

    return k(idx_grp, table, pe, gamma, beta)


def _expr_branch_tc(expr_col, w, b, pe, gamma, beta):
    BT = expr_col.shape[0]
    S, H = pe.shape
    BS = 256
    nblk_s = S // BS

    def body(e_ref, w_ref, b_ref, pe_ref, g_ref, bb_ref, o_ref):
        x = e_ref[...] * w_ref[...] + b_ref[...] + pe_ref[...]
        mu = jnp.mean(x, axis=-1, keepdims=True)
        xc = x - mu
        var = jnp.mean(xc * xc, axis=-1, keepdims=True)
        y = xc * lax.rsqrt(var + _EPS)
        o_ref[...] = y * g_ref[...] + bb_ref[...]

    return pl.pallas_call(
        body,
        grid=(BT // BS,),
        in_specs=[
            pl.BlockSpec((BS, 1), lambda i: (i, 0)),
            pl.BlockSpec((1, H), lambda i: (0, 0)),
            pl.BlockSpec((1, H), lambda i: (0, 0)),
            pl.BlockSpec((BS, H), lambda i: (i % nblk_s, 0)),
            pl.BlockSpec((1, H), lambda i: (0, 0)),
            pl.BlockSpec((1, H), lambda i: (0, 0)),
        ],
        out_specs=pl.BlockSpec((BS, H), lambda i: (i, 0)),
        out_shape=jax.ShapeDtypeStruct((BT, H), jnp.float32),
    )(expr_col, w.reshape(1, H), b.reshape(1, H), pe,
      gamma.reshape(1, H), beta.reshape(1, H))


def kernel(name, expr, name_table, w_expr, b_expr,
           gamma_name, beta_name, gamma_expr, beta_expr):
    B, S = name.shape
    V, H = name_table.shape
    pe = jnp.asarray(_pe_const(S, H))
    TOK = B * S
    NW = 32
    per_w = TOK // NW
    T = 32
    NCH = per_w // T
    idx_grp = name.reshape(NW, NCH, T)
    name_out = _name_branch_sc(idx_grp, name_table, pe, gamma_name, beta_name)
    expr_out = _expr_branch_tc(expr.reshape(TOK, 1), w_expr, b_expr, pe,
                               gamma_expr, beta_expr)
    return (name_out.reshape(B, S, H), expr_out.reshape(B, S, H))


# speedup vs baseline: 2.0579x; 2.0579x over previous
import functools
import math

import jax
import jax.numpy as jnp
import numpy as np
from jax import lax
from jax.experimental import pallas as pl
from jax.experimental.pallas import tpu as pltpu
from jax.experimental.pallas import tpu_sc as plsc

_EPS = 1e-5


def _pe_const(seq_len, dim):
    position = np.arange(seq_len, dtype=np.float32)[:, None]
    div_term = np.exp(
        np.arange(0, dim, 2, dtype=np.float32) * -(math.log(10000.0) / dim))
    pe = np.zeros((seq_len, dim), dtype=np.float32)
    pe[:, 0::2] = np.sin(position * div_term)
    pe[:, 1::2] = np.cos(position * div_term)
    return pe


def _gather_sc(idx_grp, table):
    NW, NCH, T = idx_grp.shape
    V, H = table.shape
    info = plsc.get_sparse_core_info()
    NC = info.num_cores
    per_w = NCH * T
    mesh = plsc.VectorSubcoreMesh(core_axis_name="c", subcore_axis_name="s")

    @functools.partial(
        pl.kernel,
        mesh=mesh,
        out_type=jax.ShapeDtypeStruct((NW * per_w, H), jnp.float32),
        scratch_types=[
            pltpu.VMEM((NCH, T), jnp.int32),
            pltpu.VMEM((T, H), jnp.float32),
            pltpu.VMEM((T, H), jnp.float32),
            pltpu.SemaphoreType.DMA,
            pltpu.SemaphoreType.DMA,
            pltpu.SemaphoreType.DMA,
        ],
    )
    def k(idx_hbm, table_hbm, out_hbm, idx_v, bufa, bufb, gsem, ssema, ssemb):
        wid = lax.axis_index("s") * NC + lax.axis_index("c")
        base = wid * per_w
        pltpu.sync_copy(idx_hbm.at[wid], idx_v)
        bufs = (bufa, bufb)
        ssems = (ssema, ssemb)

        def gat(j, buf):
            return pltpu.async_copy(table_hbm.at[idx_v.at[j]], buf, gsem)

        def sct(j, buf, sem):
            return pltpu.async_copy(buf, out_hbm.at[pl.ds(base + j * T, T)],
                                    sem)

        g = gat(0, bufs[0])
        s_by_buf = [None, None]
        for j in range(NCH):
            buf = bufs[j % 2]
            g.wait()
            if j + 1 < NCH:
                nxt = (j + 1) % 2
                if s_by_buf[nxt] is not None:
                    s_by_buf[nxt].wait()
                g = gat(j + 1, bufs[nxt])
            s_by_buf[j % 2] = sct(j, buf, ssems[j % 2])
        s_by_buf[(NCH - 2) % 2].wait()
        s_by_buf[(NCH - 1) % 2].wait()

    return k(idx_grp, table)


def _fused_tc(gathered, expr_col, w, b, pe, gn, bn, ge, be):
    BT, H = gathered.shape
    S = pe.shape[0]
    BS = 256
    nblk_s = S // BS

    def body(g_ref, e_ref, w_ref, b_ref, pe_ref, gn_ref, bn_ref, ge_ref,
             be_ref, no_ref, eo_ref):
        pe_blk = pe_ref[...]
        xn = g_ref[...] + pe_blk
        mu = jnp.mean(xn, axis=-1, keepdims=True)
        xc = xn - mu
        var = jnp.mean(xc * xc, axis=-1, keepdims=True)
        no_ref[...] = xc * lax.rsqrt(var + _EPS) * gn_ref[...] + bn_ref[...]
        xe = e_ref[...] * w_ref[...] + b_ref[...] + pe_blk
        mu2 = jnp.mean(xe, axis=-1, keepdims=True)
        xc2 = xe - mu2
        var2 = jnp.mean(xc2 * xc2, axis=-1, keepdims=True)
        eo_ref[...] = xc2 * lax.rsqrt(var2 + _EPS) * ge_ref[...] + be_ref[...]

    row = pl.BlockSpec((1, H), lambda i: (0, 0))
    return pl.pallas_call(
        body,
        grid=(BT // BS,),
        in_specs=[
            pl.BlockSpec((BS, H), lambda i: (i, 0)),
            pl.BlockSpec((BS, 1), lambda i: (i, 0)),
            row, row,
            pl.BlockSpec((BS, H), lambda i: (i % nblk_s, 0)),
            row, row, row, row,
        ],
        out_specs=(pl.BlockSpec((BS, H), lambda i: (i, 0)),
                   pl.BlockSpec((BS, H), lambda i: (i, 0))),
        out_shape=(jax.ShapeDtypeStruct((BT, H), jnp.float32),
                   jax.ShapeDtypeStruct((BT, H), jnp.float32)),
    )(gathered, expr_col, w.reshape(1, H), b.reshape(1, H), pe,
      gn.reshape(1, H), bn.reshape(1, H), ge.reshape(1, H), be.reshape(1, H))


def kernel(name, expr, name_table, w_expr, b_expr,
           gamma_name, beta_name, gamma_expr, beta_expr):
    B, S = name.shape
    V, H = name_table.shape
    pe = jnp.asarray(_pe_const(S, H))
    TOK = B * S
    NW = 32
    per_w = TOK // NW
    T = 32
    NCH = per_w // T
    idx_grp = name.reshape(NW, NCH, T)
    gathered = _gather_sc(idx_grp, name_table)
    name_out, expr_out = _fused_tc(gathered, expr.reshape(TOK, 1), w_expr,
                                   b_expr, pe, gamma_name, beta_name,
                                   gamma_expr, beta_expr)
    return (name_out.reshape(B, S, H), expr_out.reshape(B, S, H))


# speedup vs baseline: 2.2767x; 1.1063x over previous
import functools
import math

import jax
import jax.numpy as jnp
import numpy as np
from jax import lax
from jax.experimental import pallas as pl
from jax.experimental.pallas import tpu as pltpu
from jax.experimental.pallas import tpu_sc as plsc

_EPS = 1e-5


def _pe_const(seq_len, dim):
    position = np.arange(seq_len, dtype=np.float32)[:, None]
    div_term = np.exp(
        np.arange(0, dim, 2, dtype=np.float32) * -(math.log(10000.0) / dim))
    pe = np.zeros((seq_len, dim), dtype=np.float32)
    pe[:, 0::2] = np.sin(position * div_term)
    pe[:, 1::2] = np.cos(position * div_term)
    return pe


def _gather_sc(idx_grp, table):
    NW, NCH, T = idx_grp.shape
    V, W = table.shape
    dtype = table.dtype
    info = plsc.get_sparse_core_info()
    NC = info.num_cores
    per_w = NCH * T
    mesh = plsc.VectorSubcoreMesh(core_axis_name="c", subcore_axis_name="s")

    @functools.partial(
        pl.kernel,
        mesh=mesh,
        out_type=jax.ShapeDtypeStruct((NW * per_w, W), dtype),
        scratch_types=[
            pltpu.VMEM((NCH, T), jnp.int32),
            pltpu.VMEM((T, W), dtype),
            pltpu.VMEM((T, W), dtype),
            pltpu.SemaphoreType.DMA,
            pltpu.SemaphoreType.DMA,
            pltpu.SemaphoreType.DMA,
        ],
    )
    def k(idx_hbm, table_hbm, out_hbm, idx_v, bufa, bufb, gsem, ssema, ssemb):
        wid = lax.axis_index("s") * NC + lax.axis_index("c")
        base = wid * per_w
        pltpu.sync_copy(idx_hbm.at[wid], idx_v)
        bufs = (bufa, bufb)
        ssems = (ssema, ssemb)

        def gat(j, buf):
            return pltpu.async_copy(table_hbm.at[idx_v.at[j]], buf, gsem)

        def sct(j, buf, sem):
            return pltpu.async_copy(buf, out_hbm.at[pl.ds(base + j * T, T)],
                                    sem)

        g = gat(0, bufs[0])
        s_by_buf = [None, None]
        for j in range(NCH):
            buf = bufs[j % 2]
            g.wait()
            if j + 1 < NCH:
                nxt = (j + 1) % 2
                if s_by_buf[nxt] is not None:
                    s_by_buf[nxt].wait()
                g = gat(j + 1, bufs[nxt])
            s_by_buf[j % 2] = sct(j, buf, ssems[j % 2])
        s_by_buf[(NCH - 2) % 2].wait()
        s_by_buf[(NCH - 1) % 2].wait()

    return k(idx_grp, table)


def _fused_tc(gathered_i32, expr_col, w, b, pe, gn, bn, ge, be, nblk_b):
    BT, W = gathered_i32.shape
    S, H = pe.shape
    BS = 256
    nblk_s = S // BS

    def body(g_ref, e_ref, w_ref, b_ref, pe_ref, gn_ref, bn_ref, ge_ref,
             be_ref, no_ref, eo_ref):
        xi = g_ref[...]
        lo = lax.bitcast_convert_type(xi << 16, jnp.float32)
        hi = lax.bitcast_convert_type(xi & jnp.int32(-65536), jnp.float32)
        pe_blk = pe_ref[...].astype(jnp.float32)
        xn_lo = lo + pe_blk[:, :W]
        xn_hi = hi + pe_blk[:, W:]
        mu = ((jnp.sum(xn_lo, axis=-1, keepdims=True)
               + jnp.sum(xn_hi, axis=-1, keepdims=True)) * (1.0 / H))
        cl = xn_lo - mu
        ch = xn_hi - mu
        var = ((jnp.sum(cl * cl, axis=-1, keepdims=True)
                + jnp.sum(ch * ch, axis=-1, keepdims=True)) * (1.0 / H))
        inv = lax.rsqrt(var + _EPS)
        no_ref[:, :W] = cl * inv * gn_ref[:, :W] + bn_ref[:, :W]
        no_ref[:, W:] = ch * inv * gn_ref[:, W:] + bn_ref[:, W:]
        xe = e_ref[...] * w_ref[...] + b_ref[...] + pe_blk
        mu2 = jnp.mean(xe, axis=-1, keepdims=True)
        xc2 = xe - mu2
        var2 = jnp.mean(xc2 * xc2, axis=-1, keepdims=True)
        eo_ref[...] = xc2 * lax.rsqrt(var2 + _EPS) * ge_ref[...] + be_ref[...]

    row = pl.BlockSpec((1, H), lambda sb, bb: (0, 0))
    tok = pl.BlockSpec((BS, H), lambda sb, bb: (bb * nblk_s + sb, 0))
    return pl.pallas_call(
        body,
        grid=(nblk_s, nblk_b),
        in_specs=[
            pl.BlockSpec((BS, W), lambda sb, bb: (bb * nblk_s + sb, 0)),
            pl.BlockSpec((BS, 1), lambda sb, bb: (bb * nblk_s + sb, 0)),
            row, row,
            pl.BlockSpec((BS, H), lambda sb, bb: (sb, 0)),
            row, row, row, row,
        ],
        out_specs=(tok, tok),
        out_shape=(jax.ShapeDtypeStruct((BT, H), jnp.float32),
                   jax.ShapeDtypeStruct((BT, H), jnp.float32)),
    )(gathered_i32, expr_col, w.reshape(1, H), b.reshape(1, H), pe,
      gn.reshape(1, H), bn.reshape(1, H), ge.reshape(1, H), be.reshape(1, H))


def kernel(name, expr, name_table, w_expr, b_expr,
           gamma_name, beta_name, gamma_expr, beta_expr):
    B, S = name.shape
    V, H = name_table.shape
    W = H // 2
    pe = jnp.asarray(_pe_const(S, H).astype(jnp.bfloat16))
    tb = name_table.astype(jnp.bfloat16)
    table_packed = lax.bitcast_convert_type(
        jnp.stack([tb[:, :W], tb[:, W:]], axis=-1), jnp.int32)
    TOK = B * S
    NW = 32
    per_w = TOK // NW
    T = 64
    NCH = per_w // T
    idx_grp = name.reshape(NW, NCH, T)
    gathered = _gather_sc(idx_grp, table_packed)
    name_out, expr_out = _fused_tc(gathered, expr.reshape(TOK, 1), w_expr,
                                   b_expr, pe, gamma_name, beta_name,
                                   gamma_expr, beta_expr, B)
    return (name_out.reshape(B, S, H), expr_out.reshape(B, S, H))
